# baseline (speedup 1.0000x reference)
import functools

import jax
import jax.numpy as jnp
from jax import lax
from jax.experimental import pallas as pl
from jax.experimental.pallas import tpu as pltpu
from jax.experimental.pallas import tpu_sc as plsc

N_NODES = 10000
N_PAD = 10240
N_EDGES = 320000
F = 128
FH = F // 2

NUM_CORES = 2
NUM_SUBCORES = 16
EDGES_PER_TILE = N_EDGES // NUM_SUBCORES
CHUNK = 80
N_CHUNKS = EDGES_PER_TILE // CHUNK
FLUSH_CHUNKS = 100
EFLUSH = FLUSH_CHUNKS * CHUNK
ROWS_PER_TILE = N_PAD // NUM_SUBCORES

BN = 256
GRID = N_PAD // BN


def _hst_body(x_ref, w_ref, a_ref, h_ref, st_ref):
    h = jnp.dot(x_ref[...], w_ref[...], preferred_element_type=jnp.float32)
    h_ref[...] = h
    a1 = a_ref[0, :]
    a2 = a_ref[1, :]
    s = jnp.dot(h, a1, preferred_element_type=jnp.float32)
    t = jnp.dot(h, a2, preferred_element_type=jnp.float32)
    pad = jnp.zeros((6, BN), jnp.float32)
    st_ref[...] = jnp.concatenate([s[None, :], t[None, :], pad], axis=0)


def _stage1(x_pad, W, a2x128):
    return pl.pallas_call(
        _hst_body,
        grid=(GRID,),
        in_specs=[
            pl.BlockSpec((BN, F), lambda i: (i, 0)),
            pl.BlockSpec((F, F), lambda i: (0, 0)),
            pl.BlockSpec((2, F), lambda i: (0, 0)),
        ],
        out_specs=[
            pl.BlockSpec((BN, F), lambda i: (i, 0)),
            pl.BlockSpec((8, BN), lambda i: (0, i)),
        ],
        out_shape=[
            jax.ShapeDtypeStruct((N_PAD, F), jnp.float32),
            jax.ShapeDtypeStruct((8, N_PAD), jnp.float32),
        ],
    )(x_pad, W, a2x128)


def _sc_body(h2_hbm, s_hbm, t_hbm, src_hbm, dst_hbm, zeros_hbm,
             out_hbm, ee_hbm,
             acc, s_tab, t_tab, ee_buf, src_all, dst_all,
             rows0, rows1, rows2, rows3, dstx0, dstx1, dstx2, dstx3,
             sem_g0, sem_g1, sem_g2, sem_g3, sem_s0, sem_s1, sem_s2, sem_s3):
    cid = lax.axis_index("c")
    sid = lax.axis_index("s")

    row0 = sid * ROWS_PER_TILE
    d1 = pltpu.async_copy(s_hbm, s_tab, sem_g0)
    d2 = pltpu.async_copy(t_hbm, t_tab, sem_g1)
    d3 = pltpu.async_copy(src_hbm.at[sid], src_all, sem_g2)
    d4 = pltpu.async_copy(dst_hbm.at[sid], dst_all, sem_s0)
    d5 = pltpu.async_copy(zeros_hbm, acc.at[pl.ds(row0, ROWS_PER_TILE)], sem_s1)
    d1.wait(); d2.wait(); d3.wait(); d4.wait(); d5.wait()

    rows = (rows0, rows1, rows2, rows3)
    dstx = (dstx0, dstx1, dstx2, dstx3)
    sem_g = (sem_g0, sem_g1, sem_g2, sem_g3)
    sem_s = (sem_s0, sem_s1, sem_s2, sem_s3)
    plsc.subcore_barrier()

    def issue_gather(c, b):
        db = dstx[b]
        for k in range(CHUNK // 16):
            v = dst_all[c, pl.ds(k * 16, 16)]
            db[pl.ds(k * 16, 16)] = v + v + cid
        pltpu.async_copy(h2_hbm.at[db], rows[b], sem_g[b])

    def wait_gather(c, b):
        pltpu.make_async_copy(h2_hbm.at[dstx[b]], rows[b], sem_g[b]).wait()

    def issue_scat(c, b):
        pltpu.async_copy(rows[b], acc.at[src_all.at[c]], sem_s[b], add=True)

    def wait_scat(c, b):
        pltpu.make_async_copy(rows[b], acc.at[src_all.at[c]], sem_s[b]).wait()

    def compute(c, b):
        rb = rows[b]
        for k in range(CHUNK // 16):
            si = src_all[c, pl.ds(k * 16, 16)]
            di = dst_all[c, pl.ds(k * 16, 16)]
            sv = plsc.load_gather(s_tab, [si])
            tv = plsc.load_gather(t_tab, [di])
            e = sv + tv
            e = jnp.maximum(e, 0.2 * e)
            ev = jnp.exp(-e)
            cmod = c % FLUSH_CHUNKS
            ee_buf[pl.ds(cmod * CHUNK + k * 16, 16)] = ev
            for jj in range(16):
                eb = lax.gather(
                    ev, jnp.full((16, 1), jj, jnp.int32),
                    dimension_numbers=lax.GatherDimensionNumbers(
                        offset_dims=(), collapsed_slice_dims=(0,),
                        start_index_map=(0,)),
                    slice_sizes=(1,),
                    mode=lax.GatherScatterMode.PROMISE_IN_BOUNDS)
                j = k * 16 + jj
                for r in range(FH // 16):
                    rb[j, pl.ds(r * 16, 16)] = rb[j, pl.ds(r * 16, 16)] * eb

    issue_gather(0, 0)
    issue_gather(1, 1)
    wait_gather(0, 0)
    issue_gather(2, 2)
    compute(0, 0)
    issue_scat(0, 0)
    wait_gather(1, 1)
    issue_gather(3, 3)
    compute(1, 1)
    issue_scat(1, 1)
    wait_gather(2, 2)
    wait_scat(0, 0)
    issue_gather(4, 0)
    compute(2, 2)
    issue_scat(2, 2)

    def _quad(i, carry):
        for sub in range(4):
            cc = 4 * i + 3 + sub
            b = (3 + sub) % 4
            bn = (sub + 1) % 4
            wait_gather(cc, b)
            wait_scat(cc - 2, bn)
            issue_gather(cc + 2, bn)
            compute(cc, b)
            issue_scat(cc, b)

            @pl.when(jnp.logical_and(cc == FLUSH_CHUNKS - 1, cid == 0))
            def _flush_w1():
                pltpu.sync_copy(
                    ee_buf, ee_hbm.at[pl.ds(sid * EDGES_PER_TILE, EFLUSH)])

            @pl.when(jnp.logical_and(cc == 2 * FLUSH_CHUNKS - 1, cid == 0))
            def _flush_w2():
                pltpu.sync_copy(
                    ee_buf,
                    ee_hbm.at[pl.ds(sid * EDGES_PER_TILE + EFLUSH, EFLUSH)])
        return carry

    lax.fori_loop(0, 61, _quad, 0)
    wait_gather(247, 3)
    wait_scat(245, 1)
    issue_gather(249, 1)
    compute(247, 3)
    issue_scat(247, 3)
    for cl in (248, 249):
        b = cl % 4
        wait_gather(cl, b)
        compute(cl, b)
        issue_scat(cl, b)
    wait_scat(246, 2)
    wait_scat(247, 3)
    wait_scat(248, 0)
    wait_scat(249, 1)
    plsc.subcore_barrier()

    @pl.when(cid == 1)
    def _flush_w3():
        pltpu.sync_copy(
            ee_buf.at[pl.ds(0, EDGES_PER_TILE - 2 * EFLUSH)],
            ee_hbm.at[pl.ds(sid * EDGES_PER_TILE + 2 * EFLUSH,
                            EDGES_PER_TILE - 2 * EFLUSH)])

    n_batches = ROWS_PER_TILE // CHUNK

    def _elu_batch(k):
        r0 = row0 + k * CHUNK
        pltpu.sync_copy(acc.at[pl.ds(r0, CHUNK)], rows0)

        def _elu_row(j, carry):
            for r in range(FH // 16):
                z = rows0[j, pl.ds(r * 16, 16)]
                rows0[j, pl.ds(r * 16, 16)] = jnp.where(
                    z > 0, z, jnp.exp(z) - 1.0)
            return carry

        lax.fori_loop(0, CHUNK, _elu_row, 0)
        pltpu.sync_copy(rows0,
                        out_hbm.at[pl.ds(r0, CHUNK), pl.ds(cid * FH, FH)])

    for k in range(5):
        _elu_batch(k)
    @pl.when(sid < NUM_SUBCORES - 1)
    def _tail_batches():
        for k in range(5, n_batches):
            _elu_batch(k)


def _stage2(h2, s, t, src, dst, zeros):
    mesh = plsc.VectorSubcoreMesh(
        core_axis_name="c", subcore_axis_name="s",
        num_cores=NUM_CORES, num_subcores=NUM_SUBCORES)
    f = functools.partial(
        pl.kernel,
        out_type=[
            jax.ShapeDtypeStruct((N_NODES, F), jnp.float32),
            jax.ShapeDtypeStruct((N_EDGES,), jnp.float32),
        ],
        mesh=mesh,
        compiler_params=pltpu.CompilerParams(
            needs_layout_passes=False, use_tc_tiling_on_sc=False),
        scratch_types=[
            pltpu.VMEM_SHARED((N_PAD, FH), jnp.float32),
            pltpu.VMEM((N_PAD,), jnp.float32),
            pltpu.VMEM((N_PAD,), jnp.float32),
            pltpu.VMEM((EFLUSH,), jnp.float32),
            pltpu.VMEM((N_CHUNKS, CHUNK), jnp.int32),
            pltpu.VMEM((N_CHUNKS, CHUNK), jnp.int32),
            pltpu.VMEM((CHUNK, FH), jnp.float32),
            pltpu.VMEM((CHUNK, FH), jnp.float32),
            pltpu.VMEM((CHUNK, FH), jnp.float32),
            pltpu.VMEM((CHUNK, FH), jnp.float32),
            pltpu.VMEM((CHUNK,), jnp.int32),
            pltpu.VMEM((CHUNK,), jnp.int32),
            pltpu.VMEM((CHUNK,), jnp.int32),
            pltpu.VMEM((CHUNK,), jnp.int32),
            pltpu.SemaphoreType.DMA,
            pltpu.SemaphoreType.DMA,
            pltpu.SemaphoreType.DMA,
            pltpu.SemaphoreType.DMA,
            pltpu.SemaphoreType.DMA,
            pltpu.SemaphoreType.DMA,
            pltpu.SemaphoreType.DMA,
            pltpu.SemaphoreType.DMA,
        ],
    )(_sc_body)
    return f(h2, s, t, src, dst, zeros)


def kernel(input, non_zero, W, a):
    x = jnp.asarray(input, jnp.float32)
    a2 = jnp.asarray(a, jnp.float32).reshape(2, F)
    ei = jnp.asarray(non_zero, jnp.int32).reshape(2, NUM_SUBCORES, N_CHUNKS, CHUNK)
    src = ei[0]
    dst = ei[1]
    zeros = jnp.zeros((ROWS_PER_TILE, FH), jnp.float32)

    h, st = _stage1(x, jnp.asarray(W, jnp.float32), a2)
    s = st[0]
    t = st[1]
    h2 = h.reshape(2 * N_PAD, FH)

    out, ee = _stage2(h2, s, t, src, dst, zeros)
    return (out, ee)


# speedup vs baseline: 1.1507x; 1.1507x over previous
import functools

import jax
import jax.numpy as jnp
from jax import lax
from jax.experimental import pallas as pl
from jax.experimental.pallas import tpu as pltpu
from jax.experimental.pallas import tpu_sc as plsc

N_NODES = 10000
N_PAD = 10240
N_EDGES = 320000
F = 128
FH = F // 2

NUM_CORES = 2
NUM_SUBCORES = 16
EDGES_PER_TILE = N_EDGES // NUM_SUBCORES
CHUNK = 80
N_CHUNKS = EDGES_PER_TILE // CHUNK
HALF_CHUNKS = N_CHUNKS // 2
EHALF = EDGES_PER_TILE // 2
ROWS_PER_TILE = N_PAD // NUM_SUBCORES

BN = 256
GRID = N_PAD // BN


def _hst_body(x_ref, w_ref, a_ref, h_ref, st_ref):
    h = jnp.dot(x_ref[...], w_ref[...], preferred_element_type=jnp.float32)
    h_ref[...] = h
    a1 = a_ref[0, :]
    a2 = a_ref[1, :]
    s = jnp.dot(h, a1, preferred_element_type=jnp.float32)
    t = jnp.dot(h, a2, preferred_element_type=jnp.float32)
    pad = jnp.zeros((6, BN), jnp.float32)
    st_ref[...] = jnp.concatenate([s[None, :], t[None, :], pad], axis=0)


def _stage1(x_pad, W, a2x128):
    return pl.pallas_call(
        _hst_body,
        grid=(GRID,),
        in_specs=[
            pl.BlockSpec((BN, F), lambda i: (i, 0)),
            pl.BlockSpec((F, F), lambda i: (0, 0)),
            pl.BlockSpec((2, F), lambda i: (0, 0)),
        ],
        out_specs=[
            pl.BlockSpec((BN, F), lambda i: (i, 0)),
            pl.BlockSpec((8, BN), lambda i: (0, i)),
        ],
        out_shape=[
            jax.ShapeDtypeStruct((N_PAD, F), jnp.float32),
            jax.ShapeDtypeStruct((8, N_PAD), jnp.float32),
        ],
    )(x_pad, W, a2x128)


def _sc_body(h2_hbm, s_hbm, t_hbm, src_hbm, dst_hbm, zeros_hbm,
             out_hbm, ee_hbm,
             acc, s_tab, t_tab, ee_buf, src_all, dst_all,
             rows0, rows1, rows2, dstx0, dstx1, dstx2,
             sem_g0, sem_g1, sem_g2, sem_s0, sem_s1, sem_s2):
    cid = lax.axis_index("c")
    sid = lax.axis_index("s")

    row0 = sid * ROWS_PER_TILE
    d1 = pltpu.async_copy(s_hbm, s_tab, sem_g0)
    d2 = pltpu.async_copy(t_hbm, t_tab, sem_g1)
    d3 = pltpu.async_copy(src_hbm.at[sid], src_all, sem_g2)
    d4 = pltpu.async_copy(dst_hbm.at[sid], dst_all, sem_s0)
    d5 = pltpu.async_copy(zeros_hbm, acc.at[pl.ds(row0, ROWS_PER_TILE)], sem_s1)
    d1.wait(); d2.wait(); d3.wait(); d4.wait(); d5.wait()

    rows = (rows0, rows1, rows2)
    dstx = (dstx0, dstx1, dstx2)
    sem_g = (sem_g0, sem_g1, sem_g2)
    sem_s = (sem_s0, sem_s1, sem_s2)
    plsc.subcore_barrier()

    def issue_gather(c, b):
        db = dstx[b]
        for k in range(CHUNK // 16):
            v = dst_all[c, pl.ds(k * 16, 16)]
            db[pl.ds(k * 16, 16)] = v + v + cid
        pltpu.async_copy(h2_hbm.at[db], rows[b], sem_g[b])

    def wait_gather(c, b):
        pltpu.make_async_copy(h2_hbm.at[dstx[b]], rows[b], sem_g[b]).wait()

    def issue_scat(c, b):
        pltpu.async_copy(rows[b], acc.at[src_all.at[c]], sem_s[b], add=True)

    def wait_scat(c, b):
        pltpu.make_async_copy(rows[b], acc.at[src_all.at[c]], sem_s[b]).wait()

    def compute(c, b):
        rb = rows[b]
        for k in range(CHUNK // 16):
            si = src_all[c, pl.ds(k * 16, 16)]
            di = dst_all[c, pl.ds(k * 16, 16)]
            sv = plsc.load_gather(s_tab, [si])
            tv = plsc.load_gather(t_tab, [di])
            e = sv + tv
            e = jnp.maximum(e, 0.2 * e)
            ev = jnp.exp(-e)
            cmod = c % HALF_CHUNKS
            ee_buf[pl.ds(cmod * CHUNK + k * 16, 16)] = ev
            for jj in range(16):
                eb = lax.gather(
                    ev, jnp.full((16, 1), jj, jnp.int32),
                    dimension_numbers=lax.GatherDimensionNumbers(
                        offset_dims=(), collapsed_slice_dims=(0,),
                        start_index_map=(0,)),
                    slice_sizes=(1,),
                    mode=lax.GatherScatterMode.PROMISE_IN_BOUNDS)
                j = k * 16 + jj
                for r in range(FH // 16):
                    rb[j, pl.ds(r * 16, 16)] = rb[j, pl.ds(r * 16, 16)] * eb

    issue_gather(0, 0)
    issue_gather(1, 1)
    wait_gather(0, 0)
    compute(0, 0)
    issue_scat(0, 0)
    issue_gather(2, 2)
    wait_gather(1, 1)
    compute(1, 1)
    issue_scat(1, 1)
    wait_scat(0, 0)
    issue_gather(3, 0)

    def _triple(i, carry):
        for sub in range(3):
            cc = 3 * i + 2 + sub
            b = (2 + sub) % 3
            bn = (sub + 1) % 3
            wait_gather(cc, b)
            compute(cc, b)
            issue_scat(cc, b)
            wait_scat(cc - 1, bn)
            issue_gather(cc + 2, bn)

            @pl.when(jnp.logical_and(cc == HALF_CHUNKS - 1, cid == 0))
            def _flush_first_half():
                pltpu.sync_copy(
                    ee_buf, ee_hbm.at[pl.ds(sid * EDGES_PER_TILE, EHALF)])
        return carry

    lax.fori_loop(0, (N_CHUNKS - 4) // 3, _triple, 0)
    for cl in (N_CHUNKS - 2, N_CHUNKS - 1):
        b = cl % 3
        wait_gather(cl, b)
        compute(cl, b)
        issue_scat(cl, b)
    wait_scat(N_CHUNKS - 3, (N_CHUNKS - 3) % 3)
    wait_scat(N_CHUNKS - 2, (N_CHUNKS - 2) % 3)
    wait_scat(N_CHUNKS - 1, (N_CHUNKS - 1) % 3)
    plsc.subcore_barrier()

    @pl.when(cid == 1)
    def _flush_second_half():
        pltpu.sync_copy(
            ee_buf, ee_hbm.at[pl.ds(sid * EDGES_PER_TILE + EHALF, EHALF)])

    n_batches = ROWS_PER_TILE // CHUNK

    def _elu_batch(k):
        r0 = row0 + k * CHUNK
        pltpu.sync_copy(acc.at[pl.ds(r0, CHUNK)], rows0)

        def _elu_row(j, carry):
            for r in range(FH // 16):
                z = rows0[j, pl.ds(r * 16, 16)]
                rows0[j, pl.ds(r * 16, 16)] = jnp.where(
                    z > 0, z, jnp.exp(z) - 1.0)
            return carry

        lax.fori_loop(0, CHUNK, _elu_row, 0)
        pltpu.sync_copy(rows0,
                        out_hbm.at[pl.ds(r0, CHUNK), pl.ds(cid * FH, FH)])

    for k in range(5):
        _elu_batch(k)
    @pl.when(sid < NUM_SUBCORES - 1)
    def _tail_batches():
        for k in range(5, n_batches):
            _elu_batch(k)


def _stage2(h2, s, t, src, dst, zeros):
    mesh = plsc.VectorSubcoreMesh(
        core_axis_name="c", subcore_axis_name="s",
        num_cores=NUM_CORES, num_subcores=NUM_SUBCORES)
    f = functools.partial(
        pl.kernel,
        out_type=[
            jax.ShapeDtypeStruct((N_NODES, F), jnp.float32),
            jax.ShapeDtypeStruct((N_EDGES,), jnp.float32),
        ],
        mesh=mesh,
        compiler_params=pltpu.CompilerParams(
            needs_layout_passes=False, use_tc_tiling_on_sc=False),
        scratch_types=[
            pltpu.VMEM_SHARED((N_PAD, FH), jnp.float32),
            pltpu.VMEM((N_PAD,), jnp.float32),
            pltpu.VMEM((N_PAD,), jnp.float32),
            pltpu.VMEM((EHALF,), jnp.float32),
            pltpu.VMEM((N_CHUNKS, CHUNK), jnp.int32),
            pltpu.VMEM((N_CHUNKS, CHUNK), jnp.int32),
            pltpu.VMEM((CHUNK, FH), jnp.float32),
            pltpu.VMEM((CHUNK, FH), jnp.float32),
            pltpu.VMEM((CHUNK, FH), jnp.float32),
            pltpu.VMEM((CHUNK,), jnp.int32),
            pltpu.VMEM((CHUNK,), jnp.int32),
            pltpu.VMEM((CHUNK,), jnp.int32),
            pltpu.SemaphoreType.DMA,
            pltpu.SemaphoreType.DMA,
            pltpu.SemaphoreType.DMA,
            pltpu.SemaphoreType.DMA,
            pltpu.SemaphoreType.DMA,
            pltpu.SemaphoreType.DMA,
        ],
    )(_sc_body)
    return f(h2, s, t, src, dst, zeros)


def kernel(input, non_zero, W, a):
    x = jnp.asarray(input, jnp.float32)
    a2 = jnp.asarray(a, jnp.float32).reshape(2, F)
    nz = jnp.asarray(non_zero, jnp.int32)
    src = nz[0].reshape(NUM_SUBCORES, N_CHUNKS, CHUNK)
    dst = nz[1].reshape(NUM_SUBCORES, N_CHUNKS, CHUNK)
    zeros = jnp.zeros((ROWS_PER_TILE, FH), jnp.float32)

    h, st = _stage1(x, jnp.asarray(W, jnp.float32), a2)
    s = st[0]
    t = st[1]
    h2 = h.reshape(2 * N_PAD, FH)

    out, ee = _stage2(h2, s, t, src, dst, zeros)
    return (out, ee)


# speedup vs baseline: 1.1655x; 1.0129x over previous
import functools

import jax
import jax.numpy as jnp
from jax import lax
from jax.experimental import pallas as pl
from jax.experimental.pallas import tpu as pltpu
from jax.experimental.pallas import tpu_sc as plsc

N_NODES = 10000
N_PAD = 10240
N_EDGES = 320000
F = 128
FH = F // 2

NUM_CORES = 2
NUM_SUBCORES = 16
EDGES_PER_TILE = N_EDGES // NUM_SUBCORES
CHUNK = 80
N_CHUNKS = EDGES_PER_TILE // CHUNK
HALF_CHUNKS = N_CHUNKS // 2
EHALF = EDGES_PER_TILE // 2
ROWS_PER_TILE = N_PAD // NUM_SUBCORES

BN = 256
GRID = N_PAD // BN


def _hst_body(x_ref, w_ref, a_ref, h_ref, st_ref):
    h = jnp.dot(x_ref[...], w_ref[...], preferred_element_type=jnp.float32)
    h_ref[...] = h
    a1 = a_ref[0, :]
    a2 = a_ref[1, :]
    s = jnp.dot(h, a1, preferred_element_type=jnp.float32)
    t = jnp.dot(h, a2, preferred_element_type=jnp.float32)
    pad = jnp.zeros((6, BN), jnp.float32)
    st_ref[...] = jnp.concatenate([s[None, :], t[None, :], pad], axis=0)


def _stage1(x_pad, W, a2x128):
    return pl.pallas_call(
        _hst_body,
        grid=(GRID,),
        in_specs=[
            pl.BlockSpec((BN, F), lambda i: (i, 0)),
            pl.BlockSpec((F, F), lambda i: (0, 0)),
            pl.BlockSpec((2, F), lambda i: (0, 0)),
        ],
        out_specs=[
            pl.BlockSpec((BN, F), lambda i: (i, 0)),
            pl.BlockSpec((8, BN), lambda i: (0, i)),
        ],
        out_shape=[
            jax.ShapeDtypeStruct((N_PAD, F), jnp.float32),
            jax.ShapeDtypeStruct((8, N_PAD), jnp.float32),
        ],
    )(x_pad, W, a2x128)


def _sc_body(h2_hbm, s_hbm, t_hbm, src_hbm, dst_hbm, zeros_hbm,
             out_hbm, ee_hbm,
             acc, s_tab, t_tab, ee_buf, src_all, dst_all,
             rows0, rows1, rows2, dstx0, dstx1, dstx2,
             sem_g0, sem_g1, sem_g2, sem_s0, sem_s1, sem_s2):
    cid = lax.axis_index("c")
    sid = lax.axis_index("s")

    row0 = sid * ROWS_PER_TILE
    d1 = pltpu.async_copy(s_hbm, s_tab, sem_g0)
    d2 = pltpu.async_copy(t_hbm, t_tab, sem_g1)
    d3 = pltpu.async_copy(src_hbm.at[sid], src_all, sem_g2)
    d4 = pltpu.async_copy(dst_hbm.at[sid], dst_all, sem_s0)
    d5 = pltpu.async_copy(zeros_hbm, acc.at[pl.ds(row0, ROWS_PER_TILE)], sem_s1)
    d1.wait(); d2.wait(); d3.wait(); d4.wait(); d5.wait()

    rows = (rows0, rows1, rows2)
    dstx = (dstx0, dstx1, dstx2)
    sem_g = (sem_g0, sem_g1, sem_g2)
    sem_s = (sem_s0, sem_s1, sem_s2)
    plsc.subcore_barrier()

    def issue_gather(c, b):
        db = dstx[b]
        for k in range(CHUNK // 16):
            v = dst_all[c, pl.ds(k * 16, 16)]
            db[pl.ds(k * 16, 16)] = v + v + cid
        pltpu.async_copy(h2_hbm.at[db], rows[b], sem_g[b])

    def wait_gather(c, b):
        pltpu.make_async_copy(h2_hbm.at[dstx[b]], rows[b], sem_g[b]).wait()

    def issue_scat(c, b):
        pltpu.async_copy(rows[b], acc.at[src_all.at[c]], sem_s[b], add=True)

    def wait_scat(c, b):
        pltpu.make_async_copy(rows[b], acc.at[src_all.at[c]], sem_s[b]).wait()

    def compute(c, b):
        rb = rows[b]
        for k in range(CHUNK // 16):
            si = src_all[c, pl.ds(k * 16, 16)]
            di = dst_all[c, pl.ds(k * 16, 16)]
            sv = plsc.load_gather(s_tab, [si])
            tv = plsc.load_gather(t_tab, [di])
            e = sv + tv
            e = jnp.maximum(e, 0.2 * e)
            ev = jnp.exp(-e)
            cmod = c % HALF_CHUNKS
            ee_buf[pl.ds(cmod * CHUNK + k * 16, 16)] = ev
            for jj in range(16):
                eb = lax.gather(
                    ev, jnp.full((16, 1), jj, jnp.int32),
                    dimension_numbers=lax.GatherDimensionNumbers(
                        offset_dims=(), collapsed_slice_dims=(0,),
                        start_index_map=(0,)),
                    slice_sizes=(1,),
                    mode=lax.GatherScatterMode.PROMISE_IN_BOUNDS)
                j = k * 16 + jj
                for r in range(FH // 16):
                    rb[j, pl.ds(r * 16, 16)] = rb[j, pl.ds(r * 16, 16)] * eb

    issue_gather(0, 0)
    issue_gather(1, 1)
    wait_gather(0, 0)
    compute(0, 0)
    issue_scat(0, 0)
    issue_gather(2, 2)
    wait_gather(1, 1)
    compute(1, 1)
    issue_scat(1, 1)
    wait_scat(0, 0)
    issue_gather(3, 0)

    def _triple(i, carry):
        for sub in range(3):
            cc = 3 * i + 2 + sub
            b = (2 + sub) % 3
            bn = (sub + 1) % 3
            wait_gather(cc, b)
            compute(cc, b)
            issue_scat(cc, b)
            wait_scat(cc - 1, bn)
            issue_gather(cc + 2, bn)

            @pl.when(jnp.logical_and(cc == HALF_CHUNKS - 1, cid == 0))
            def _flush_first_half():
                pltpu.sync_copy(
                    ee_buf, ee_hbm.at[pl.ds(sid * EDGES_PER_TILE, EHALF)])
        return carry

    lax.fori_loop(0, (N_CHUNKS - 4) // 3, _triple, 0)
    for cl in (N_CHUNKS - 2, N_CHUNKS - 1):
        b = cl % 3
        wait_gather(cl, b)
        compute(cl, b)
        issue_scat(cl, b)
    wait_scat(N_CHUNKS - 3, (N_CHUNKS - 3) % 3)
    wait_scat(N_CHUNKS - 2, (N_CHUNKS - 2) % 3)
    wait_scat(N_CHUNKS - 1, (N_CHUNKS - 1) % 3)
    plsc.subcore_barrier()

    @pl.when(cid == 1)
    def _flush_second_half():
        pltpu.sync_copy(
            ee_buf, ee_hbm.at[pl.ds(sid * EDGES_PER_TILE + EHALF, EHALF)])

    n_batches = ROWS_PER_TILE // CHUNK
    n_valid = 5

    def ein(k, b):
        pltpu.async_copy(acc.at[pl.ds(row0 + k * CHUNK, CHUNK)],
                         rows[b], sem_g[b])

    def ein_wait(k, b):
        pltpu.make_async_copy(acc.at[pl.ds(row0 + k * CHUNK, CHUNK)],
                              rows[b], sem_g[b]).wait()

    def eout(k, b):
        pltpu.async_copy(
            rows[b],
            out_hbm.at[pl.ds(row0 + k * CHUNK, CHUNK), pl.ds(cid * FH, FH)],
            sem_s[b])

    def eout_wait(k, b):
        pltpu.make_async_copy(
            rows[b],
            out_hbm.at[pl.ds(row0 + k * CHUNK, CHUNK), pl.ds(cid * FH, FH)],
            sem_s[b]).wait()

    def _elu(b):
        rb = rows[b]

        def _elu_row(j, carry):
            for r in range(FH // 16):
                z = rb[j, pl.ds(r * 16, 16)]
                rb[j, pl.ds(r * 16, 16)] = jnp.where(z > 0, z, jnp.exp(z) - 1.0)
            return carry

        lax.fori_loop(0, CHUNK, _elu_row, 0)

    ein(0, 0)
    ein(1, 1)
    for k in range(n_batches):
        b = k % 3
        ein_wait(k, b)
        _elu(b)
        if k < n_valid:
            eout(k, b)
        else:
            @pl.when(sid < NUM_SUBCORES - 1)
            def _eo(k=k, b=b):
                eout(k, b)
        if k + 2 < n_batches:
            kw = k - 1
            if kw >= 0:
                if kw < n_valid:
                    eout_wait(kw, kw % 3)
                else:
                    @pl.when(sid < NUM_SUBCORES - 1)
                    def _ew(kw=kw):
                        eout_wait(kw, kw % 3)
            ein(k + 2, (k + 2) % 3)
    for kw in range(n_batches - 3, n_batches):
        if kw < n_valid:
            eout_wait(kw, kw % 3)
        else:
            @pl.when(sid < NUM_SUBCORES - 1)
            def _ew2(kw=kw):
                eout_wait(kw, kw % 3)


def _stage2(h2, s, t, src, dst, zeros):
    mesh = plsc.VectorSubcoreMesh(
        core_axis_name="c", subcore_axis_name="s",
        num_cores=NUM_CORES, num_subcores=NUM_SUBCORES)
    f = functools.partial(
        pl.kernel,
        out_type=[
            jax.ShapeDtypeStruct((N_NODES, F), jnp.float32),
            jax.ShapeDtypeStruct((N_EDGES,), jnp.float32),
        ],
        mesh=mesh,
        compiler_params=pltpu.CompilerParams(
            needs_layout_passes=False, use_tc_tiling_on_sc=False),
        scratch_types=[
            pltpu.VMEM_SHARED((N_PAD, FH), jnp.float32),
            pltpu.VMEM((N_PAD,), jnp.float32),
            pltpu.VMEM((N_PAD,), jnp.float32),
            pltpu.VMEM((EHALF,), jnp.float32),
            pltpu.VMEM((N_CHUNKS, CHUNK), jnp.int32),
            pltpu.VMEM((N_CHUNKS, CHUNK), jnp.int32),
            pltpu.VMEM((CHUNK, FH), jnp.float32),
            pltpu.VMEM((CHUNK, FH), jnp.float32),
            pltpu.VMEM((CHUNK, FH), jnp.float32),
            pltpu.VMEM((CHUNK,), jnp.int32),
            pltpu.VMEM((CHUNK,), jnp.int32),
            pltpu.VMEM((CHUNK,), jnp.int32),
            pltpu.SemaphoreType.DMA,
            pltpu.SemaphoreType.DMA,
            pltpu.SemaphoreType.DMA,
            pltpu.SemaphoreType.DMA,
            pltpu.SemaphoreType.DMA,
            pltpu.SemaphoreType.DMA,
        ],
    )(_sc_body)
    return f(h2, s, t, src, dst, zeros)


def kernel(input, non_zero, W, a):
    x = jnp.asarray(input, jnp.float32)
    a2 = jnp.asarray(a, jnp.float32).reshape(2, F)
    nz = jnp.asarray(non_zero, jnp.int32)
    src = nz[0].reshape(NUM_SUBCORES, N_CHUNKS, CHUNK)
    dst = nz[1].reshape(NUM_SUBCORES, N_CHUNKS, CHUNK)
    zeros = jnp.zeros((ROWS_PER_TILE, FH), jnp.float32)

    h, st = _stage1(x, jnp.asarray(W, jnp.float32), a2)
    s = st[0]
    t = st[1]
    h2 = h.reshape(2 * N_PAD, FH)

    out, ee = _stage2(h2, s, t, src, dst, zeros)
    return (out, ee)


# speedup vs baseline: 1.2047x; 1.0336x over previous
import functools

import jax
import jax.numpy as jnp
from jax import lax
from jax.experimental import pallas as pl
from jax.experimental.pallas import tpu as pltpu
from jax.experimental.pallas import tpu_sc as plsc

N_NODES = 10000
N_PAD = 10240
N_EDGES = 320000
F = 128
FH = F // 2

NUM_CORES = 2
NUM_SUBCORES = 16
EDGES_PER_TILE = N_EDGES // NUM_SUBCORES
CHUNK = 80
N_CHUNKS = EDGES_PER_TILE // CHUNK
FLUSH_CHUNKS = 100
EFLUSH = FLUSH_CHUNKS * CHUNK
ROWS_PER_TILE = N_PAD // NUM_SUBCORES

BN = 256
GRID = N_PAD // BN


def _hst_body(x_ref, w_ref, a_ref, h_ref, st_ref):
    h = jnp.dot(x_ref[...], w_ref[...], preferred_element_type=jnp.float32)
    h_ref[...] = h.astype(jnp.bfloat16)
    a1 = a_ref[0, :]
    a2 = a_ref[1, :]
    s = jnp.dot(h, a1, preferred_element_type=jnp.float32)
    t = jnp.dot(h, a2, preferred_element_type=jnp.float32)
    pad = jnp.zeros((6, BN), jnp.float32)
    st_ref[...] = jnp.concatenate([s[None, :], t[None, :], pad], axis=0)


def _stage1(x_pad, W, a2x128):
    return pl.pallas_call(
        _hst_body,
        grid=(GRID,),
        in_specs=[
            pl.BlockSpec((BN, F), lambda i: (i, 0)),
            pl.BlockSpec((F, F), lambda i: (0, 0)),
            pl.BlockSpec((2, F), lambda i: (0, 0)),
        ],
        out_specs=[
            pl.BlockSpec((BN, F), lambda i: (i, 0)),
            pl.BlockSpec((8, BN), lambda i: (0, i)),
        ],
        out_shape=[
            jax.ShapeDtypeStruct((N_PAD, F), jnp.bfloat16),
            jax.ShapeDtypeStruct((8, N_PAD), jnp.float32),
        ],
    )(x_pad, W, a2x128)


def _sc_body(h2_hbm, s_hbm, t_hbm, comb_hbm, zeros_hbm,
             out_hbm, ee_hbm,
             acc, s_tab, t_tab, ee_buf, comb_all,
             rows0, rows1, rows2, rowsb0, rowsb1, rowsb2,
             dstx0, dstx1, dstx2, srcb0, srcb1, srcb2,
             sem_g0, sem_g1, sem_g2, sem_s0, sem_s1, sem_s2):
    cid = lax.axis_index("c")
    sid = lax.axis_index("s")

    row0 = sid * ROWS_PER_TILE
    d1 = pltpu.async_copy(s_hbm.at[pl.ds(0, N_NODES)], s_tab, sem_g0)
    d2 = pltpu.async_copy(t_hbm.at[pl.ds(0, N_NODES)], t_tab, sem_g1)
    d3 = pltpu.async_copy(comb_hbm.at[sid], comb_all, sem_g2)
    d5 = pltpu.async_copy(zeros_hbm, acc.at[pl.ds(row0, ROWS_PER_TILE)], sem_s1)
    d1.wait(); d2.wait(); d3.wait(); d5.wait()

    rows = (rows0, rows1, rows2)
    rowsb = (rowsb0, rowsb1, rowsb2)
    dstx = (dstx0, dstx1, dstx2)
    srcb = (srcb0, srcb1, srcb2)
    sem_g = (sem_g0, sem_g1, sem_g2)
    sem_s = (sem_s0, sem_s1, sem_s2)
    plsc.subcore_barrier()

    def issue_gather(c, b):
        db = dstx[b]
        sb = srcb[b]
        for k in range(CHUNK // 16):
            v = comb_all[c, pl.ds(k * 16, 16)]
            dv = v >> 16
            sb[pl.ds(k * 16, 16)] = v & 0xFFFF
            db[pl.ds(k * 16, 16)] = dv + dv + cid
        pltpu.async_copy(h2_hbm.at[db], rowsb[b], sem_g[b])

    def wait_gather(c, b):
        pltpu.make_async_copy(h2_hbm.at[dstx[b]], rowsb[b], sem_g[b]).wait()

    def issue_scat(c, b):
        pltpu.async_copy(rows[b], acc.at[srcb[b]], sem_s[b], add=True)

    def wait_scat(c, b):
        pltpu.make_async_copy(rows[b], acc.at[srcb[b]], sem_s[b]).wait()

    def compute(c, b):
        rb = rows[b]
        for k in range(CHUNK // 16):
            si = srcb[b][pl.ds(k * 16, 16)]
            di = dstx[b][pl.ds(k * 16, 16)] >> 1
            sv = plsc.load_gather(s_tab, [si])
            tv = plsc.load_gather(t_tab, [di])
            e = sv + tv
            e = jnp.maximum(e, 0.2 * e)
            ev = jnp.exp(-e)
            cmod = c % FLUSH_CHUNKS
            ee_buf[pl.ds(cmod * CHUNK + k * 16, 16)] = ev
            rbb = rowsb[b]
            for jj in range(16):
                eb = lax.gather(
                    ev, jnp.full((16, 1), jj, jnp.int32),
                    dimension_numbers=lax.GatherDimensionNumbers(
                        offset_dims=(), collapsed_slice_dims=(0,),
                        start_index_map=(0,)),
                    slice_sizes=(1,),
                    mode=lax.GatherScatterMode.PROMISE_IN_BOUNDS)
                j = k * 16 + jj
                for g in range(FH // 32):
                    ab = rbb[j, pl.ds(g * 32, 32)]
                    av, bv = plsc.unpack(ab, format=plsc.PackFormat.INTERLEAVED)
                    rb[j, pl.ds(g * 32, 16)] = av * eb
                    rb[j, pl.ds(g * 32 + 16, 16)] = bv * eb

    issue_gather(0, 0)
    issue_gather(1, 1)
    wait_gather(0, 0)
    compute(0, 0)
    issue_scat(0, 0)
    issue_gather(2, 2)
    wait_gather(1, 1)
    compute(1, 1)
    issue_scat(1, 1)
    wait_scat(0, 0)
    issue_gather(3, 0)

    def _triple(i, carry):
        for sub in range(3):
            cc = 3 * i + 2 + sub
            b = (2 + sub) % 3
            bn = (sub + 1) % 3
            wait_gather(cc, b)
            compute(cc, b)
            issue_scat(cc, b)
            wait_scat(cc - 1, bn)
            issue_gather(cc + 2, bn)

            @pl.when(jnp.logical_and(cc == FLUSH_CHUNKS - 1, cid == 0))
            def _flush_w1():
                pltpu.sync_copy(
                    ee_buf, ee_hbm.at[pl.ds(sid * EDGES_PER_TILE, EFLUSH)])

            @pl.when(jnp.logical_and(cc == 2 * FLUSH_CHUNKS - 1, cid == 0))
            def _flush_w2():
                pltpu.sync_copy(
                    ee_buf,
                    ee_hbm.at[pl.ds(sid * EDGES_PER_TILE + EFLUSH, EFLUSH)])
        return carry

    lax.fori_loop(0, (N_CHUNKS - 4) // 3, _triple, 0)
    for cl in (N_CHUNKS - 2, N_CHUNKS - 1):
        b = cl % 3
        wait_gather(cl, b)
        compute(cl, b)
        issue_scat(cl, b)
    wait_scat(N_CHUNKS - 3, (N_CHUNKS - 3) % 3)
    wait_scat(N_CHUNKS - 2, (N_CHUNKS - 2) % 3)
    wait_scat(N_CHUNKS - 1, (N_CHUNKS - 1) % 3)
    plsc.subcore_barrier()

    @pl.when(cid == 1)
    def _flush_w3():
        pltpu.sync_copy(
            ee_buf.at[pl.ds(0, EDGES_PER_TILE - 2 * EFLUSH)],
            ee_hbm.at[pl.ds(sid * EDGES_PER_TILE + 2 * EFLUSH,
                            EDGES_PER_TILE - 2 * EFLUSH)])

    n_batches = ROWS_PER_TILE // CHUNK
    n_valid = 5

    def ein(k, b):
        pltpu.async_copy(acc.at[pl.ds(row0 + k * CHUNK, CHUNK)],
                         rows[b], sem_g[b])

    def ein_wait(k, b):
        pltpu.make_async_copy(acc.at[pl.ds(row0 + k * CHUNK, CHUNK)],
                              rows[b], sem_g[b]).wait()

    def eout(k, b):
        pltpu.async_copy(
            rows[b],
            out_hbm.at[pl.ds(row0 + k * CHUNK, CHUNK), pl.ds(cid * FH, FH)],
            sem_s[b])

    def eout_wait(k, b):
        pltpu.make_async_copy(
            rows[b],
            out_hbm.at[pl.ds(row0 + k * CHUNK, CHUNK), pl.ds(cid * FH, FH)],
            sem_s[b]).wait()

    def _elu(b):
        rb = rows[b]

        iota16 = lax.iota(jnp.int32, 16)

        def _elu_row(j, carry):
            jsp = jnp.full((16,), j, jnp.int32)
            zs = []
            for g in range(FH // 16):
                c = iota16 + (16 * g)
                pos = (((c >> 5) << 5) + ((c & 1) << 4)
                       + ((c & 31) >> 1))
                zs.append(plsc.load_gather(rb, [jsp, pos]))
            for g in range(FH // 16):
                z = zs[g]
                rb[j, pl.ds(g * 16, 16)] = jnp.where(z > 0, z, jnp.exp(z) - 1.0)
            return carry

        lax.fori_loop(0, CHUNK, _elu_row, 0)

    ein(0, 0)
    ein(1, 1)
    for k in range(n_batches):
        b = k % 3
        ein_wait(k, b)
        _elu(b)
        if k < n_valid:
            eout(k, b)
        else:
            @pl.when(sid < NUM_SUBCORES - 1)
            def _eo(k=k, b=b):
                eout(k, b)
        if k + 2 < n_batches:
            kw = k - 1
            if kw >= 0:
                if kw < n_valid:
                    eout_wait(kw, kw % 3)
                else:
                    @pl.when(sid < NUM_SUBCORES - 1)
                    def _ew(kw=kw):
                        eout_wait(kw, kw % 3)
            ein(k + 2, (k + 2) % 3)
    for kw in range(n_batches - 3, n_batches):
        if kw < n_valid:
            eout_wait(kw, kw % 3)
        else:
            @pl.when(sid < NUM_SUBCORES - 1)
            def _ew2(kw=kw):
                eout_wait(kw, kw % 3)


def _stage2(h2, s, t, comb, zeros):
    mesh = plsc.VectorSubcoreMesh(
        core_axis_name="c", subcore_axis_name="s",
        num_cores=NUM_CORES, num_subcores=NUM_SUBCORES)
    f = functools.partial(
        pl.kernel,
        out_type=[
            jax.ShapeDtypeStruct((N_NODES, F), jnp.float32),
            jax.ShapeDtypeStruct((N_EDGES,), jnp.float32),
        ],
        mesh=mesh,
        compiler_params=pltpu.CompilerParams(
            needs_layout_passes=False, use_tc_tiling_on_sc=False),
        scratch_types=[
            pltpu.VMEM_SHARED((N_PAD, FH), jnp.float32),
            pltpu.VMEM((N_NODES,), jnp.float32),
            pltpu.VMEM((N_NODES,), jnp.float32),
            pltpu.VMEM((EFLUSH,), jnp.float32),
            pltpu.VMEM((N_CHUNKS, CHUNK), jnp.int32),
            pltpu.VMEM((CHUNK, FH), jnp.float32),
            pltpu.VMEM((CHUNK, FH), jnp.float32),
            pltpu.VMEM((CHUNK, FH), jnp.float32),
            pltpu.VMEM((CHUNK, FH), jnp.bfloat16),
            pltpu.VMEM((CHUNK, FH), jnp.bfloat16),
            pltpu.VMEM((CHUNK, FH), jnp.bfloat16),
            pltpu.VMEM((CHUNK,), jnp.int32),
            pltpu.VMEM((CHUNK,), jnp.int32),
            pltpu.VMEM((CHUNK,), jnp.int32),
            pltpu.VMEM((CHUNK,), jnp.int32),
            pltpu.VMEM((CHUNK,), jnp.int32),
            pltpu.VMEM((CHUNK,), jnp.int32),
            pltpu.SemaphoreType.DMA,
            pltpu.SemaphoreType.DMA,
            pltpu.SemaphoreType.DMA,
            pltpu.SemaphoreType.DMA,
            pltpu.SemaphoreType.DMA,
            pltpu.SemaphoreType.DMA,
        ],
    )(_sc_body)
    return f(h2, s, t, comb, zeros)


def kernel(input, non_zero, W, a):
    x = jnp.asarray(input, jnp.float32)
    a2 = jnp.asarray(a, jnp.float32).reshape(2, F)
    nz = jnp.asarray(non_zero, jnp.int32)
    comb = (nz[0] | (nz[1] << 16)).reshape(NUM_SUBCORES, N_CHUNKS, CHUNK)
    zeros = jnp.zeros((ROWS_PER_TILE, FH), jnp.float32)

    h_bf, st = _stage1(x, jnp.asarray(W, jnp.float32), a2)
    s = st[0]
    t = st[1]
    h2 = h_bf.reshape(2 * N_PAD, FH)

    out, ee = _stage2(h2, s, t, comb, zeros)
    return (out, ee)
